# baseline (speedup 1.0000x reference)
import functools

import jax
import jax.numpy as jnp
from jax import lax
from jax.experimental import pallas as pl
from jax.experimental.pallas import tpu as pltpu
from jax.experimental.pallas import tpu_sc as plsc

NC = 2
NS = 16
LANES = 16

EC = 80
ECC = 80


def _pre_body(x_ref, w_ref, root_ref, b_ref, t0_ref, td_ref, xr_ref):
    xb = x_ref[...]
    w0 = w_ref[0]
    wd = w_ref[1] - w0
    t0_ref[...] = jnp.dot(xb, w0, preferred_element_type=jnp.float32)
    td_ref[...] = jnp.dot(xb, wd, preferred_element_type=jnp.float32)
    xr_ref[...] = (
        jnp.dot(xb, root_ref[...], preferred_element_type=jnp.float32)
        + b_ref[...]
    )


def _pre_tc(x, W, root, b):
    n, d = x.shape
    bn = 1000
    grid = (n // bn,)
    t0, td, xr = pl.pallas_call(
        _pre_body,
        grid=grid,
        in_specs=[
            pl.BlockSpec((bn, d), lambda i: (i, 0)),
            pl.BlockSpec((2, d, d), lambda i: (0, 0, 0)),
            pl.BlockSpec((d, d), lambda i: (0, 0)),
            pl.BlockSpec((1, d), lambda i: (0, 0)),
        ],
        out_specs=[
            pl.BlockSpec((bn, d), lambda i: (i, 0)),
            pl.BlockSpec((bn, d), lambda i: (i, 0)),
            pl.BlockSpec((bn, d), lambda i: (i, 0)),
        ],
        out_shape=[
            jax.ShapeDtypeStruct((n, d), jnp.float32),
            jax.ShapeDtypeStruct((n, d), jnp.float32),
            jax.ShapeDtypeStruct((n, d), jnp.float32),
        ],
    )(x, W, root, b.reshape(1, d))
    return t0, td, xr


def _post_body(acc_ref, cnt_ref, xr_ref, o_ref, *, final):
    s = acc_ref[0] + acc_ref[1]
    cnt = jnp.sum(cnt_ref[0] + cnt_ref[1], axis=1, keepdims=True)
    h = s / jnp.maximum(cnt, 1.0) + xr_ref[...]
    if final:
        m = jnp.max(h, axis=1, keepdims=True)
        e = jnp.exp(h - m)
        lse = jnp.log(jnp.sum(e, axis=1, keepdims=True)) + m
        o_ref[...] = h - lse
    else:
        o_ref[...] = jnp.where(h > 0.0, h, jnp.exp(h) - 1.0)


def _post_tc(acc, cnt, xr, final):
    n, d = xr.shape
    bn = 1000
    grid = (n // bn,)
    return pl.pallas_call(
        functools.partial(_post_body, final=final),
        grid=grid,
        in_specs=[
            pl.BlockSpec((NC, bn, d), lambda i: (0, i, 0)),
            pl.BlockSpec((NC, bn, d), lambda i: (0, i, 0)),
            pl.BlockSpec((bn, d), lambda i: (i, 0)),
        ],
        out_specs=pl.BlockSpec((bn, d), lambda i: (i, 0)),
        out_shape=jax.ShapeDtypeStruct((n, d), jnp.float32),
    )(acc, cnt, xr)


def _mid_body(acc_ref, cnt_ref, xr_ref, w_ref, root_ref, b_ref,
              t0_ref, td_ref, xr2_ref):
    s = acc_ref[0] + acc_ref[1]
    cnt = jnp.sum(cnt_ref[0] + cnt_ref[1], axis=1, keepdims=True)
    h = s / jnp.maximum(cnt, 1.0) + xr_ref[...]
    h = jnp.where(h > 0.0, h, jnp.exp(h) - 1.0)
    w0 = w_ref[0]
    wd = w_ref[1] - w0
    t0_ref[...] = jnp.dot(h, w0, preferred_element_type=jnp.float32)
    td_ref[...] = jnp.dot(h, wd, preferred_element_type=jnp.float32)
    xr2_ref[...] = (
        jnp.dot(h, root_ref[...], preferred_element_type=jnp.float32)
        + b_ref[...]
    )


def _mid_tc(acc, cnt, xr, W, root, b):
    n, d = xr.shape
    bn = 1000
    grid = (n // bn,)
    return pl.pallas_call(
        _mid_body,
        grid=grid,
        in_specs=[
            pl.BlockSpec((NC, bn, d), lambda i: (0, i, 0)),
            pl.BlockSpec((NC, bn, d), lambda i: (0, i, 0)),
            pl.BlockSpec((bn, d), lambda i: (i, 0)),
            pl.BlockSpec((2, d, d), lambda i: (0, 0, 0)),
            pl.BlockSpec((d, d), lambda i: (0, 0)),
            pl.BlockSpec((1, d), lambda i: (0, 0)),
        ],
        out_specs=[
            pl.BlockSpec((bn, d), lambda i: (i, 0)),
            pl.BlockSpec((bn, d), lambda i: (i, 0)),
            pl.BlockSpec((bn, d), lambda i: (i, 0)),
        ],
        out_shape=[
            jax.ShapeDtypeStruct((n, d), jnp.float32),
            jax.ShapeDtypeStruct((n, d), jnp.float32),
            jax.ShapeDtypeStruct((n, d), jnp.float32),
        ],
    )(acc, cnt, xr, W, root, b.reshape(1, d))


def _npad(n):
    return ((n + NS * 128 - 1) // (NS * 128)) * NS * 128


def _make_cnt_sc(n, e):
    kpt = e // (NC * NS * ECC)
    npad = _npad(n)
    rps = npad // NS
    mesh = plsc.VectorSubcoreMesh(core_axis_name="c", subcore_axis_name="s")

    @functools.partial(
        pl.kernel,
        out_type=jax.ShapeDtypeStruct((NC, npad, 128), jnp.float32),
        mesh=mesh,
        scratch_types=[
            pltpu.VMEM((kpt, ECC), jnp.int32),
            pltpu.VMEM((ECC, 128), jnp.float32),
            pltpu.SemaphoreType.DMA,
            pltpu.VMEM_SHARED((npad, 128), jnp.float32),
        ],
    )
    def cnt_kernel(dst_hbm, ones_hbm, zeros_hbm, cnt_hbm,
                   dst_v, ones_v, sem, acc_sh):
        c = lax.axis_index("c")
        s = lax.axis_index("s")
        wid = c * NS + s
        pltpu.sync_copy(dst_hbm.at[wid], dst_v)
        pltpu.sync_copy(ones_hbm, ones_v)
        for q in range(rps // ECC):
            pltpu.sync_copy(zeros_hbm,
                            acc_sh.at[pl.ds(s * rps + q * ECC, ECC)])
        plsc.subcore_barrier()

        def chunk(k, carry):
            pltpu.sync_copy(ones_v, acc_sh.at[dst_v.at[k]], add=True)
            return carry
        lax.fori_loop(0, kpt, chunk, 0)

        plsc.subcore_barrier()
        pltpu.sync_copy(acc_sh.at[pl.ds(s * rps, rps)],
                        cnt_hbm.at[c, pl.ds(s * rps, rps)])

    return cnt_kernel


def _make_edge_sc(n, e):
    kpt = e // (NC * NS * EC)
    npad = _npad(n)
    rps = npad // NS
    mesh = plsc.VectorSubcoreMesh(core_axis_name="c", subcore_axis_name="s")

    ept = kpt * EC

    @functools.partial(
        pl.kernel,
        out_type=jax.ShapeDtypeStruct((NC, npad, 128), jnp.float32),
        mesh=mesh,
        scratch_types=[
            pltpu.VMEM((4, EC), jnp.int32),
            pltpu.VMEM((4, EC), jnp.int32),
            pltpu.VMEM((2, EC), jnp.float32),
            pltpu.VMEM((2 * EC, 128), jnp.float32),
            pltpu.VMEM((2 * EC, 128), jnp.float32),
            pltpu.SemaphoreType.DMA,
            pltpu.SemaphoreType.DMA,
            pltpu.SemaphoreType.DMA,
            pltpu.SemaphoreType.DMA,
            pltpu.SemaphoreType.DMA,
            pltpu.SemaphoreType.DMA,
            pltpu.VMEM_SHARED((npad, 128), jnp.float32),
        ],
    )
    def edge_kernel(t0_hbm, td_hbm, src_hbm, dst_hbm, u_hbm, out_hbm,
                    src_v, dst_v, u_v, x0_v, xd_v,
                    sem_r, sem_u, sem_d, sem_g0, sem_g1, sem_s, acc_sh):
        c = lax.axis_index("c")
        s = lax.axis_index("s")
        wid = c * NS + s

        zeros16 = jnp.zeros((LANES,), jnp.float32)

        def zrow(r, carry):
            for f in range(8):
                x0_v[r, pl.ds(f * LANES, LANES)] = zeros16
            return carry
        lax.fori_loop(0, EC, zrow, 0)
        for q in range(rps // EC):
            pltpu.sync_copy(x0_v.at[pl.ds(0, EC)],
                            acc_sh.at[pl.ds(s * rps + q * EC, EC)])
        plsc.subcore_barrier()

        def issue_src(k):
            pltpu.async_copy(src_hbm.at[wid, pl.ds(k, 1)],
                             src_v.at[pl.ds(lax.rem(k, 4), 1)], sem_r)

        def wait_src(k):
            pltpu.make_async_copy(
                src_hbm.at[wid, pl.ds(k, 1)],
                src_v.at[pl.ds(lax.rem(k, 4), 1)], sem_r).wait()

        def issue_idx(k):
            pltpu.async_copy(u_hbm.at[wid, k], u_v.at[lax.rem(k, 2)], sem_u)
            pltpu.async_copy(dst_hbm.at[wid, pl.ds(k, 1)],
                             dst_v.at[pl.ds(lax.rem(k, 4), 1)], sem_d)

        def issue_gathers(k):
            rsel = pl.ds(lax.rem(k, 2) * EC, EC)
            idx = src_v.at[lax.rem(k, 4)]
            pltpu.async_copy(t0_hbm.at[idx], x0_v.at[rsel], sem_g0)
            pltpu.async_copy(td_hbm.at[idx], xd_v.at[rsel], sem_g1)

        def wait_scatter():
            pltpu.make_async_copy(
                x0_v.at[pl.ds(0, EC)], acc_sh.at[pl.ds(0, EC)], sem_s).wait()

        issue_src(0)
        wait_src(0)
        issue_gathers(0)
        issue_src(1)
        issue_idx(0)

        def steady(k, carry):
            ku = lax.rem(k, 2)
            kd = lax.rem(k, 4)
            rbase = ku * EC

            idx = src_v.at[kd]
            rsel = pl.ds(rbase, EC)
            pltpu.make_async_copy(t0_hbm.at[idx], x0_v.at[rsel], sem_g0).wait()
            pltpu.make_async_copy(td_hbm.at[idx], xd_v.at[rsel], sem_g1).wait()
            pltpu.make_async_copy(
                u_hbm.at[wid, k], u_v.at[ku], sem_u).wait()
            pltpu.make_async_copy(
                dst_hbm.at[wid, pl.ds(k, 1)],
                dst_v.at[pl.ds(kd, 1)], sem_d).wait()

            @pl.when(k >= 1)
            def _():
                wait_scatter()

            @pl.when(k + 1 < kpt)
            def _():
                issue_idx(k + 1)
                wait_src(k + 1)

                @pl.when(k + 2 < kpt)
                def _():
                    issue_src(k + 2)
                issue_gathers(k + 1)

            def do_group(base, lane0):
                u16 = u_v[ku, pl.ds(base, LANES)]
                u16 = jnp.minimum(jnp.maximum(u16, 0.0), 1.0)
                for lane in range(lane0, LANES):
                    el = rbase + base + lane
                    us = lax.gather(
                        u16,
                        jnp.full((LANES, 1), lane, jnp.int32),
                        lax.GatherDimensionNumbers(
                            offset_dims=(), collapsed_slice_dims=(0,),
                            start_index_map=(0,)),
                        slice_sizes=(1,),
                        mode=lax.GatherScatterMode.PROMISE_IN_BOUNDS)
                    for f in range(8):
                        a = x0_v[el, pl.ds(f * LANES, LANES)]
                        dd = xd_v[el, pl.ds(f * LANES, LANES)]
                        x0_v[el, pl.ds(f * LANES, LANES)] = a + us * dd

            def group(g, carry2):
                do_group(g * LANES, 0)
                return carry2
            lax.fori_loop(0, EC // LANES, group, 0)
            if EC % LANES:
                do_group(EC - LANES, LANES - (EC % LANES))

            pltpu.async_copy(
                x0_v.at[pl.ds(rbase, EC)],
                acc_sh.at[dst_v.at[kd]], sem_s, add=True)
            return carry
        lax.fori_loop(0, kpt, steady, 0)

        wait_scatter()
        plsc.subcore_barrier()
        pltpu.sync_copy(acc_sh.at[pl.ds(s * rps, rps)],
                        out_hbm.at[c, pl.ds(s * rps, rps)])

    return edge_kernel


def kernel(x, edge_index, edge_attr, W1, root1, b1, W2, root2, b2):
    n = x.shape[0]
    e = edge_index.shape[1]
    nw = NC * NS
    kpt = e // (nw * EC)
    kptc = e // (nw * ECC)

    src1 = edge_index[0].reshape(nw, kpt, EC)
    dst3 = edge_index[1].reshape(nw, kpt, EC)
    dstc = edge_index[1].reshape(nw, kptc, ECC)
    u3 = edge_attr[:, 0].reshape(nw, kpt, EC)

    cnt_sc = _make_cnt_sc(n, e)
    edge_sc = _make_edge_sc(n, e)

    onehot = jnp.zeros((ECC, 128), jnp.float32).at[:, 0].set(1.0)
    zrows = jnp.zeros((ECC, 128), jnp.float32)
    cnt = cnt_sc(dstc, onehot, zrows)

    t01, td1, xr1 = _pre_tc(x, W1, root1, b1)
    acc1 = edge_sc(t01, td1, src1, dst3, u3)
    t02, td2, xr2 = _mid_tc(acc1, cnt, xr1, W2, root2, b2)
    acc2 = edge_sc(t02, td2, src1, dst3, u3)
    return _post_tc(acc2, cnt, xr2, final=True)


# speedup vs baseline: 1.0268x; 1.0268x over previous
import functools

import jax
import jax.numpy as jnp
from jax import lax
from jax.experimental import pallas as pl
from jax.experimental.pallas import tpu as pltpu
from jax.experimental.pallas import tpu_sc as plsc

NC = 2
NS = 16
LANES = 16

EC = 80
ECC = 80


def _pre_body(x_ref, w_ref, root_ref, b_ref, t0_ref, td_ref, xr_ref):
    xb = x_ref[...]
    w0 = w_ref[0]
    wd = w_ref[1] - w0
    t0_ref[...] = jnp.dot(xb, w0, preferred_element_type=jnp.float32)
    td_ref[...] = jnp.dot(xb, wd, preferred_element_type=jnp.float32)
    xr_ref[...] = (
        jnp.dot(xb, root_ref[...], preferred_element_type=jnp.float32)
        + b_ref[...]
    )


def _pre_tc(x, W, root, b):
    n, d = x.shape
    bn = 1000
    grid = (n // bn,)
    t0, td, xr = pl.pallas_call(
        _pre_body,
        grid=grid,
        in_specs=[
            pl.BlockSpec((bn, d), lambda i: (i, 0)),
            pl.BlockSpec((2, d, d), lambda i: (0, 0, 0)),
            pl.BlockSpec((d, d), lambda i: (0, 0)),
            pl.BlockSpec((1, d), lambda i: (0, 0)),
        ],
        out_specs=[
            pl.BlockSpec((bn, d), lambda i: (i, 0)),
            pl.BlockSpec((bn, d), lambda i: (i, 0)),
            pl.BlockSpec((bn, d), lambda i: (i, 0)),
        ],
        out_shape=[
            jax.ShapeDtypeStruct((n, d), jnp.float32),
            jax.ShapeDtypeStruct((n, d), jnp.float32),
            jax.ShapeDtypeStruct((n, d), jnp.float32),
        ],
    )(x, W, root, b.reshape(1, d))
    return t0, td, xr


def _post_body(acc_ref, cnt_ref, xr_ref, o_ref, *, final):
    s = acc_ref[0] + acc_ref[1]
    cnt = jnp.sum(cnt_ref[0] + cnt_ref[1], axis=1, keepdims=True)
    h = s / jnp.maximum(cnt, 1.0) + xr_ref[...]
    if final:
        m = jnp.max(h, axis=1, keepdims=True)
        e = jnp.exp(h - m)
        lse = jnp.log(jnp.sum(e, axis=1, keepdims=True)) + m
        o_ref[...] = h - lse
    else:
        o_ref[...] = jnp.where(h > 0.0, h, jnp.exp(h) - 1.0)


def _post_tc(acc, cnt, xr, final):
    n, d = xr.shape
    bn = 1000
    grid = (n // bn,)
    return pl.pallas_call(
        functools.partial(_post_body, final=final),
        grid=grid,
        in_specs=[
            pl.BlockSpec((NC, bn, d), lambda i: (0, i, 0)),
            pl.BlockSpec((NC, bn, d), lambda i: (0, i, 0)),
            pl.BlockSpec((bn, d), lambda i: (i, 0)),
        ],
        out_specs=pl.BlockSpec((bn, d), lambda i: (i, 0)),
        out_shape=jax.ShapeDtypeStruct((n, d), jnp.float32),
    )(acc, cnt, xr)


def _mid_body(acc_ref, cnt_ref, xr_ref, w_ref, root_ref, b_ref,
              t0_ref, td_ref, xr2_ref):
    s = acc_ref[0] + acc_ref[1]
    cnt = jnp.sum(cnt_ref[0] + cnt_ref[1], axis=1, keepdims=True)
    h = s / jnp.maximum(cnt, 1.0) + xr_ref[...]
    h = jnp.where(h > 0.0, h, jnp.exp(h) - 1.0)
    w0 = w_ref[0]
    wd = w_ref[1] - w0
    t0_ref[...] = jnp.dot(h, w0, preferred_element_type=jnp.float32)
    td_ref[...] = jnp.dot(h, wd, preferred_element_type=jnp.float32)
    xr2_ref[...] = (
        jnp.dot(h, root_ref[...], preferred_element_type=jnp.float32)
        + b_ref[...]
    )


def _mid_tc(acc, cnt, xr, W, root, b):
    n, d = xr.shape
    bn = 1000
    grid = (n // bn,)
    return pl.pallas_call(
        _mid_body,
        grid=grid,
        in_specs=[
            pl.BlockSpec((NC, bn, d), lambda i: (0, i, 0)),
            pl.BlockSpec((NC, bn, d), lambda i: (0, i, 0)),
            pl.BlockSpec((bn, d), lambda i: (i, 0)),
            pl.BlockSpec((2, d, d), lambda i: (0, 0, 0)),
            pl.BlockSpec((d, d), lambda i: (0, 0)),
            pl.BlockSpec((1, d), lambda i: (0, 0)),
        ],
        out_specs=[
            pl.BlockSpec((bn, d), lambda i: (i, 0)),
            pl.BlockSpec((bn, d), lambda i: (i, 0)),
            pl.BlockSpec((bn, d), lambda i: (i, 0)),
        ],
        out_shape=[
            jax.ShapeDtypeStruct((n, d), jnp.float32),
            jax.ShapeDtypeStruct((n, d), jnp.float32),
            jax.ShapeDtypeStruct((n, d), jnp.float32),
        ],
    )(acc, cnt, xr, W, root, b.reshape(1, d))


def _npad(n):
    return ((n + NS * 128 - 1) // (NS * 128)) * NS * 128


def _make_cnt_sc(n, e):
    kpt = e // (NC * NS * ECC)
    npad = _npad(n)
    rps = npad // NS
    mesh = plsc.VectorSubcoreMesh(core_axis_name="c", subcore_axis_name="s")

    @functools.partial(
        pl.kernel,
        out_type=jax.ShapeDtypeStruct((NC, npad, 128), jnp.float32),
        mesh=mesh,
        scratch_types=[
            pltpu.VMEM((kpt, ECC), jnp.int32),
            pltpu.VMEM((ECC, 128), jnp.float32),
            pltpu.VMEM_SHARED((npad, 128), jnp.float32),
        ],
    )
    def cnt_kernel(dst_hbm, cnt_hbm, dst_v, ones_v, acc_sh):
        c = lax.axis_index("c")
        s = lax.axis_index("s")
        wid = c * NS + s
        pltpu.sync_copy(dst_hbm.at[wid], dst_v)
        zeros16 = jnp.zeros((LANES,), jnp.float32)
        cntvec = jnp.where(
            lax.iota(jnp.int32, LANES) == 0,
            jnp.float32(1.0), jnp.float32(0.0))

        def zrow(r, carry):
            for f in range(8):
                ones_v[r, pl.ds(f * LANES, LANES)] = zeros16
            return carry
        lax.fori_loop(0, ECC, zrow, 0)
        for q in range(rps // ECC):
            pltpu.sync_copy(ones_v, acc_sh.at[pl.ds(s * rps + q * ECC, ECC)])

        def orow(r, carry):
            ones_v[r, pl.ds(0, LANES)] = cntvec
            return carry
        lax.fori_loop(0, ECC, orow, 0)
        plsc.subcore_barrier()

        def chunk(k, carry):
            pltpu.sync_copy(ones_v, acc_sh.at[dst_v.at[k]], add=True)
            return carry
        lax.fori_loop(0, kpt, chunk, 0)

        plsc.subcore_barrier()
        pltpu.sync_copy(acc_sh.at[pl.ds(s * rps, rps)],
                        cnt_hbm.at[c, pl.ds(s * rps, rps)])

    return cnt_kernel


def _make_edge_sc(n, e):
    kpt = e // (NC * NS * EC)
    npad = _npad(n)
    rps = npad // NS
    mesh = plsc.VectorSubcoreMesh(core_axis_name="c", subcore_axis_name="s")

    ept = kpt * EC

    @functools.partial(
        pl.kernel,
        out_type=jax.ShapeDtypeStruct((NC, npad, 128), jnp.float32),
        mesh=mesh,
        scratch_types=[
            pltpu.VMEM((4, EC), jnp.int32),
            pltpu.VMEM((4, EC), jnp.int32),
            pltpu.VMEM((2, EC), jnp.float32),
            pltpu.VMEM((2 * EC, 128), jnp.float32),
            pltpu.VMEM((2 * EC, 128), jnp.float32),
            pltpu.SemaphoreType.DMA,
            pltpu.SemaphoreType.DMA,
            pltpu.SemaphoreType.DMA,
            pltpu.SemaphoreType.DMA,
            pltpu.SemaphoreType.DMA,
            pltpu.SemaphoreType.DMA,
            pltpu.VMEM_SHARED((npad, 128), jnp.float32),
        ],
    )
    def edge_kernel(t0_hbm, td_hbm, src_hbm, dst_hbm, u_hbm, out_hbm,
                    src_v, dst_v, u_v, x0_v, xd_v,
                    sem_r, sem_u, sem_d, sem_g0, sem_g1, sem_s, acc_sh):
        c = lax.axis_index("c")
        s = lax.axis_index("s")
        wid = c * NS + s

        zeros16 = jnp.zeros((LANES,), jnp.float32)

        def zrow(r, carry):
            for f in range(8):
                x0_v[r, pl.ds(f * LANES, LANES)] = zeros16
            return carry
        lax.fori_loop(0, EC, zrow, 0)
        for q in range(rps // EC):
            pltpu.sync_copy(x0_v.at[pl.ds(0, EC)],
                            acc_sh.at[pl.ds(s * rps + q * EC, EC)])
        plsc.subcore_barrier()

        def issue_src(k):
            pltpu.async_copy(src_hbm.at[wid, pl.ds(k, 1)],
                             src_v.at[pl.ds(lax.rem(k, 4), 1)], sem_r)

        def wait_src(k):
            pltpu.make_async_copy(
                src_hbm.at[wid, pl.ds(k, 1)],
                src_v.at[pl.ds(lax.rem(k, 4), 1)], sem_r).wait()

        def issue_idx(k):
            pltpu.async_copy(u_hbm.at[wid, k], u_v.at[lax.rem(k, 2)], sem_u)
            pltpu.async_copy(dst_hbm.at[wid, pl.ds(k, 1)],
                             dst_v.at[pl.ds(lax.rem(k, 4), 1)], sem_d)

        def issue_gathers(k):
            rsel = pl.ds(lax.rem(k, 2) * EC, EC)
            idx = src_v.at[lax.rem(k, 4)]
            pltpu.async_copy(t0_hbm.at[idx], x0_v.at[rsel], sem_g0)
            pltpu.async_copy(td_hbm.at[idx], xd_v.at[rsel], sem_g1)

        def wait_scatter():
            pltpu.make_async_copy(
                x0_v.at[pl.ds(0, EC)], acc_sh.at[pl.ds(0, EC)], sem_s).wait()

        issue_src(0)
        wait_src(0)
        issue_gathers(0)
        issue_src(1)
        issue_idx(0)

        def steady(k, carry):
            ku = lax.rem(k, 2)
            kd = lax.rem(k, 4)
            rbase = ku * EC

            idx = src_v.at[kd]
            rsel = pl.ds(rbase, EC)
            pltpu.make_async_copy(t0_hbm.at[idx], x0_v.at[rsel], sem_g0).wait()
            pltpu.make_async_copy(td_hbm.at[idx], xd_v.at[rsel], sem_g1).wait()
            pltpu.make_async_copy(
                u_hbm.at[wid, k], u_v.at[ku], sem_u).wait()
            pltpu.make_async_copy(
                dst_hbm.at[wid, pl.ds(k, 1)],
                dst_v.at[pl.ds(kd, 1)], sem_d).wait()

            @pl.when(k >= 1)
            def _():
                wait_scatter()

            @pl.when(k + 1 < kpt)
            def _():
                issue_idx(k + 1)
                wait_src(k + 1)

                @pl.when(k + 2 < kpt)
                def _():
                    issue_src(k + 2)
                issue_gathers(k + 1)

            def do_group(base, lane0):
                u16 = u_v[ku, pl.ds(base, LANES)]
                u16 = jnp.minimum(jnp.maximum(u16, 0.0), 1.0)
                for lane in range(lane0, LANES):
                    el = rbase + base + lane
                    us = lax.gather(
                        u16,
                        jnp.full((LANES, 1), lane, jnp.int32),
                        lax.GatherDimensionNumbers(
                            offset_dims=(), collapsed_slice_dims=(0,),
                            start_index_map=(0,)),
                        slice_sizes=(1,),
                        mode=lax.GatherScatterMode.PROMISE_IN_BOUNDS)
                    for f in range(8):
                        a = x0_v[el, pl.ds(f * LANES, LANES)]
                        dd = xd_v[el, pl.ds(f * LANES, LANES)]
                        x0_v[el, pl.ds(f * LANES, LANES)] = a + us * dd

            def group(g, carry2):
                do_group(g * LANES, 0)
                return carry2
            lax.fori_loop(0, EC // LANES, group, 0)
            if EC % LANES:
                do_group(EC - LANES, LANES - (EC % LANES))

            pltpu.async_copy(
                x0_v.at[pl.ds(rbase, EC)],
                acc_sh.at[dst_v.at[kd]], sem_s, add=True)
            return carry
        lax.fori_loop(0, kpt, steady, 0)

        wait_scatter()
        plsc.subcore_barrier()
        pltpu.sync_copy(acc_sh.at[pl.ds(s * rps, rps)],
                        out_hbm.at[c, pl.ds(s * rps, rps)])

    return edge_kernel


def kernel(x, edge_index, edge_attr, W1, root1, b1, W2, root2, b2):
    n = x.shape[0]
    e = edge_index.shape[1]
    nw = NC * NS
    kpt = e // (nw * EC)
    kptc = e // (nw * ECC)

    src1 = edge_index[0].reshape(nw, kpt, EC)
    dst3 = edge_index[1].reshape(nw, kpt, EC)
    dstc = edge_index[1].reshape(nw, kptc, ECC)
    u3 = edge_attr[:, 0].reshape(nw, kpt, EC)

    cnt_sc = _make_cnt_sc(n, e)
    edge_sc = _make_edge_sc(n, e)

    cnt = cnt_sc(dstc)

    t01, td1, xr1 = _pre_tc(x, W1, root1, b1)
    acc1 = edge_sc(t01, td1, src1, dst3, u3)
    t02, td2, xr2 = _mid_tc(acc1, cnt, xr1, W2, root2, b2)
    acc2 = edge_sc(t02, td2, src1, dst3, u3)
    return _post_tc(acc2, cnt, xr2, final=True)
